# Initial kernel scaffold; baseline (speedup 1.0000x reference)
import functools

import jax
import jax.numpy as jnp
from jax import lax
from jax.experimental import pallas as pl
from jax.experimental.pallas import tpu as pltpu
from jax.experimental.pallas import tpu_sc as plsc

N_NODES = 10000
N_EDGES = 320000
D = 128

NC = 2
NS = 16
NW = NC * NS

CHUNK = 125
EROWS = N_EDGES // CHUNK
ROWS_PER_TILE = EROWS // NW
NODES_PER_TILE = N_NODES // NS
DEG_W = 16


def _sc_agg_body(with_deg, *refs):
    if with_deg:
        (x_hbm, src_hbm, dst_hbm, out_hbm, deg_hbm,
         src_v, dst_v, rows_v, zero_v, ones_v, zdeg_v, agg_sh, deg_sh, sem) = refs
    else:
        (x_hbm, src_hbm, dst_hbm, out_hbm,
         src_v, dst_v, rows_v, zero_v, agg_sh, sem) = refs

    cid = lax.axis_index("c")
    sid = lax.axis_index("s")
    wid = cid * NS + sid

    zf32 = jnp.zeros((16,), jnp.float32)

    @pl.loop(0, CHUNK)
    def _(i):
        for j in range(D // 16):
            zero_v[i, pl.ds(16 * j, 16)] = zf32

    if with_deg:
        of32 = jnp.full((16,), 1.0, jnp.float32)

        @pl.loop(0, CHUNK)
        def _(i):
            ones_v[i, :] = of32
            zdeg_v[i, :] = zf32

    node_base = sid * NODES_PER_TILE
    for k in range(NODES_PER_TILE // CHUNK):
        pltpu.sync_copy(zero_v, agg_sh.at[pl.ds(node_base + k * CHUNK, CHUNK)])
        if with_deg:
            pltpu.sync_copy(zdeg_v, deg_sh.at[pl.ds(node_base + k * CHUNK, CHUNK)])

    row_base = wid * ROWS_PER_TILE
    pltpu.sync_copy(src_hbm.at[pl.ds(row_base, ROWS_PER_TILE)], src_v)
    pltpu.sync_copy(dst_hbm.at[pl.ds(row_base, ROWS_PER_TILE)], dst_v)

    plsc.subcore_barrier()

    @pl.loop(0, ROWS_PER_TILE)
    def _(c):
        pltpu.async_copy(x_hbm.at[src_v.at[c]], rows_v, sem).wait()
        pltpu.sync_copy(rows_v, agg_sh.at[dst_v.at[c]], add=True)
        if with_deg:
            pltpu.sync_copy(ones_v, deg_sh.at[dst_v.at[c]], add=True)

    plsc.subcore_barrier()

    pltpu.sync_copy(agg_sh.at[pl.ds(node_base, NODES_PER_TILE)],
                    out_hbm.at[cid, pl.ds(node_base, NODES_PER_TILE)])
    if with_deg:
        pltpu.sync_copy(deg_sh.at[pl.ds(node_base, NODES_PER_TILE)],
                        deg_hbm.at[cid, pl.ds(node_base, NODES_PER_TILE)])


def _make_sc_agg(with_deg):
    mesh = plsc.VectorSubcoreMesh(core_axis_name="c", subcore_axis_name="s",
                                  num_cores=NC, num_subcores=NS)
    out_type = [jax.ShapeDtypeStruct((NC, N_NODES, D), jnp.float32)]
    scratch = [
        pltpu.VMEM((ROWS_PER_TILE, CHUNK), jnp.int32),
        pltpu.VMEM((ROWS_PER_TILE, CHUNK), jnp.int32),
        pltpu.VMEM((CHUNK, D), jnp.float32),
        pltpu.VMEM((CHUNK, D), jnp.float32),
    ]
    if with_deg:
        out_type.append(jax.ShapeDtypeStruct((NC, N_NODES, DEG_W), jnp.float32))
        scratch += [
            pltpu.VMEM((CHUNK, DEG_W), jnp.float32),
            pltpu.VMEM((CHUNK, DEG_W), jnp.float32),
        ]
    scratch.append(pltpu.VMEM_SHARED((N_NODES, D), jnp.float32))
    if with_deg:
        scratch.append(pltpu.VMEM_SHARED((N_NODES, DEG_W), jnp.float32))
    scratch.append(pltpu.SemaphoreType.DMA)

    return pl.kernel(
        functools.partial(_sc_agg_body, with_deg),
        out_type=tuple(out_type) if len(out_type) > 1 else out_type[0],
        mesh=mesh,
        scratch_types=tuple(scratch),
        name="sc_sage_agg_deg" if with_deg else "sc_sage_agg",
    )


_sc_agg_deg = _make_sc_agg(True)
_sc_agg = _make_sc_agg(False)


def _tc1_body(p_ref, dp_ref, x_ref, wl_ref, b_ref, wr_ref, h_ref, dr_ref):
    agg = p_ref[0] + p_ref[1]
    deg = dp_ref[0, :, 0:1] + dp_ref[1, :, 0:1]
    r = 1.0 / jnp.maximum(deg, 1.0)
    a = agg * r
    h = (jnp.dot(a, wl_ref[...], preferred_element_type=jnp.float32)
         + b_ref[...]
         + jnp.dot(x_ref[...], wr_ref[...], preferred_element_type=jnp.float32))
    h_ref[...] = jnp.maximum(h, 0.0)
    dr_ref[...] = jnp.broadcast_to(r, (N_NODES, DEG_W))


def _tc2_body(p_ref, dr_ref, x_ref, wl_ref, b_ref, wr_ref, o_ref):
    agg = p_ref[0] + p_ref[1]
    a = agg * dr_ref[:, 0:1]
    o_ref[...] = (jnp.dot(a, wl_ref[...], preferred_element_type=jnp.float32)
                  + b_ref[...]
                  + jnp.dot(x_ref[...], wr_ref[...], preferred_element_type=jnp.float32))


_tc1 = pl.pallas_call(
    _tc1_body,
    out_shape=(jax.ShapeDtypeStruct((N_NODES, D), jnp.float32),
               jax.ShapeDtypeStruct((N_NODES, DEG_W), jnp.float32)),
)

_tc2 = pl.pallas_call(
    _tc2_body,
    out_shape=jax.ShapeDtypeStruct((N_NODES, D), jnp.float32),
)


@jax.jit
def kernel(x, edge_index, W1l, b1, W1r, W2l, b2, W2r):
    src = edge_index[0].reshape(EROWS, CHUNK)
    dst = edge_index[1].reshape(EROWS, CHUNK)

    agg1, degp = _sc_agg_deg(x, src, dst)
    h, degr = _tc1(agg1, degp, x, W1l.T, b1.reshape(1, D), W1r.T)
    (agg2,) = _sc_agg(h, src, dst)
    out = _tc2(agg2, degr, h, W2l.T, b2.reshape(1, D), W2r.T)
    return out


# speedup vs baseline: 6.5007x; 6.5007x over previous
import functools

import jax
import jax.numpy as jnp
from jax import lax
from jax.experimental import pallas as pl
from jax.experimental.pallas import tpu as pltpu
from jax.experimental.pallas import tpu_sc as plsc

N_NODES = 10000
N_EDGES = 320000
D = 128
DH = D // 2

NC = 2
NS = 16

CHUNK = 125
EROWS = N_EDGES // CHUNK
ROWS_PER_TILE = EROWS // NS
DEG_SPLIT = ROWS_PER_TILE // 2
NODES_PER_TILE = N_NODES // NS
WB_ROWS = NODES_PER_TILE // 8 * 8
DEG_W = 16


def _edge_loop(x_hbm, src_v, dst_v, rows_v, ones_v, agg_sh, deg_sh, sem,
               deg_lo, deg_hi):
    @pl.loop(0, ROWS_PER_TILE)
    def _(c):
        pltpu.async_copy(x_hbm.at[src_v.at[c]], rows_v, sem).wait()
        pltpu.sync_copy(rows_v, agg_sh.at[dst_v.at[c]], add=True)

        @pl.when((c >= deg_lo) & (c < deg_hi))
        def _():
            pltpu.sync_copy(ones_v, deg_sh.at[dst_v.at[c]], add=True)


def _sc_agg_body(x0_hbm, x1_hbm, src_hbm, dst_hbm, out_hbm, deg_hbm,
                 src_v, dst_v, rows_v, zero_v, ones_v, zdeg_v,
                 agg_sh, deg_sh, sem):
    cid = lax.axis_index("c")
    sid = lax.axis_index("s")

    zf32 = jnp.zeros((16,), jnp.float32)
    of32 = jnp.full((16,), 1.0, jnp.float32)

    @pl.loop(0, CHUNK)
    def _(i):
        for j in range(DH // 16):
            zero_v[i, pl.ds(16 * j, 16)] = zf32
        ones_v[i, :] = of32
        zdeg_v[i, :] = zf32

    node_base = sid * NODES_PER_TILE
    for k in range(NODES_PER_TILE // CHUNK):
        pltpu.sync_copy(zero_v, agg_sh.at[pl.ds(node_base + k * CHUNK, CHUNK)])
        pltpu.sync_copy(zdeg_v, deg_sh.at[pl.ds(node_base + k * CHUNK, CHUNK)])

    row_base = sid * ROWS_PER_TILE
    pltpu.sync_copy(src_hbm.at[pl.ds(row_base, ROWS_PER_TILE)], src_v)
    pltpu.sync_copy(dst_hbm.at[pl.ds(row_base, ROWS_PER_TILE)], dst_v)

    plsc.subcore_barrier()

    @pl.when(cid == 0)
    def _():
        _edge_loop(x0_hbm, src_v, dst_v, rows_v, ones_v, agg_sh, deg_sh, sem,
                   0, DEG_SPLIT)

    @pl.when(cid == 1)
    def _():
        _edge_loop(x1_hbm, src_v, dst_v, rows_v, ones_v, agg_sh, deg_sh, sem,
                   DEG_SPLIT, ROWS_PER_TILE)

    plsc.subcore_barrier()

    wb_base = sid * WB_ROWS
    pltpu.sync_copy(agg_sh.at[pl.ds(wb_base, WB_ROWS)],
                    out_hbm.at[cid, pl.ds(wb_base, WB_ROWS)])
    pltpu.sync_copy(deg_sh.at[pl.ds(wb_base, WB_ROWS)],
                    deg_hbm.at[cid, pl.ds(wb_base, WB_ROWS)])

    @pl.when(sid == 0)
    def _():
        tail = N_NODES - NS * WB_ROWS
        pltpu.sync_copy(agg_sh.at[pl.ds(NS * WB_ROWS, tail)],
                        out_hbm.at[cid, pl.ds(NS * WB_ROWS, tail)])
        pltpu.sync_copy(deg_sh.at[pl.ds(NS * WB_ROWS, tail)],
                        deg_hbm.at[cid, pl.ds(NS * WB_ROWS, tail)])


_sc_agg = pl.kernel(
    _sc_agg_body,
    out_type=(jax.ShapeDtypeStruct((NC, N_NODES, DH), jnp.float32),
              jax.ShapeDtypeStruct((NC, N_NODES, DEG_W), jnp.float32)),
    mesh=plsc.VectorSubcoreMesh(core_axis_name="c", subcore_axis_name="s",
                                num_cores=NC, num_subcores=NS),
    scratch_types=(
        pltpu.VMEM((ROWS_PER_TILE, CHUNK), jnp.int32),
        pltpu.VMEM((ROWS_PER_TILE, CHUNK), jnp.int32),
        pltpu.VMEM((CHUNK, DH), jnp.float32),
        pltpu.VMEM((CHUNK, DH), jnp.float32),
        pltpu.VMEM((CHUNK, DEG_W), jnp.float32),
        pltpu.VMEM((CHUNK, DEG_W), jnp.float32),
        pltpu.VMEM_SHARED((N_NODES, DH), jnp.float32),
        pltpu.VMEM_SHARED((N_NODES, DEG_W), jnp.float32),
        pltpu.SemaphoreType.DMA,
    ),
    compiler_params=pltpu.CompilerParams(use_tc_tiling_on_sc=False),
    name="sc_sage_agg",
)


def _tc_body(relu, p_ref, dp_ref, x_ref, wl_ref, b_ref, wr_ref, o_ref):
    agg = jnp.concatenate((p_ref[0], p_ref[1]), axis=1)
    deg = dp_ref[0, :, 0:1] + dp_ref[1, :, 0:1]
    a = agg / jnp.maximum(deg, 1.0)
    o = (jnp.dot(a, wl_ref[...], preferred_element_type=jnp.float32)
         + b_ref[...]
         + jnp.dot(x_ref[...], wr_ref[...], preferred_element_type=jnp.float32))
    o_ref[...] = jnp.maximum(o, 0.0) if relu else o


def _make_tc(relu):
    return pl.pallas_call(
        functools.partial(_tc_body, relu),
        out_shape=jax.ShapeDtypeStruct((N_NODES, D), jnp.float32),
    )


_tc1 = _make_tc(True)
_tc2 = _make_tc(False)


@jax.jit
def kernel(x, edge_index, W1l, b1, W1r, W2l, b2, W2r):
    src = edge_index[0].reshape(EROWS, CHUNK)
    dst = edge_index[1].reshape(EROWS, CHUNK)

    agg1, degp = _sc_agg(x[:, :DH], x[:, DH:], src, dst)
    h = _tc1(agg1, degp, x, W1l.T, b1.reshape(1, D), W1r.T)
    agg2, degp2 = _sc_agg(h[:, :DH], h[:, DH:], src, dst)
    out = _tc2(agg2, degp2, h, W2l.T, b2.reshape(1, D), W2r.T)
    return out


# speedup vs baseline: 10.1526x; 1.5618x over previous
import functools

import jax
import jax.numpy as jnp
from jax import lax
from jax.experimental import pallas as pl
from jax.experimental.pallas import tpu as pltpu
from jax.experimental.pallas import tpu_sc as plsc

N_NODES = 10000
N_EDGES = 320000
D = 128
DH = D // 2

NC = 2
NS = 16

CHUNK = 125
EROWS = N_EDGES // CHUNK
ROWS_PER_TILE = EROWS // NS
DEG_SPLIT = ROWS_PER_TILE // 2
NODES_PER_TILE = N_NODES // NS
WB_ROWS = NODES_PER_TILE // 8 * 8
DEG_W = 16


NBUF = 2


def _prime(x_hbm, src_v, bufs, gsems):
    for b in range(NBUF):
        pltpu.async_copy(x_hbm.at[src_v.at[b]], bufs[b], gsems[b])


def _edge_loop(x_hbm, src_v, dst_v, bufs, ones_v, agg_sh, deg_sh,
               gsems, ssems, deg_lo, deg_hi):
    @pl.loop(0, ROWS_PER_TILE, step=NBUF)
    def _(c):
        for b in range(NBUF):
            k = c + b
            pltpu.make_async_copy(x_hbm.at[src_v.at[k]], bufs[b],
                                  gsems[b]).wait()
            sdesc = pltpu.async_copy(bufs[b], agg_sh.at[dst_v.at[k]],
                                     ssems[b], add=True)

            @pl.when((k >= deg_lo) & (k < deg_hi))
            def _():
                pltpu.sync_copy(ones_v, deg_sh.at[dst_v.at[k]], add=True)

            sdesc.wait()

            @pl.when(k + NBUF < ROWS_PER_TILE)
            def _():
                pltpu.async_copy(x_hbm.at[src_v.at[k + NBUF]], bufs[b],
                                 gsems[b])


def _sc_agg_body(x0_hbm, x1_hbm, src_hbm, dst_hbm, out_hbm, deg_hbm,
                 src_v, dst_v, b0, b1, b2, b3, zero_v, ones_v, zdeg_v,
                 agg_sh, deg_sh, g0, g1, g2, g3, s0, s1, s2, s3):
    bufs = (b0, b1, b2, b3)
    gsems = (g0, g1, g2, g3)
    ssems = (s0, s1, s2, s3)
    cid = lax.axis_index("c")
    sid = lax.axis_index("s")

    zf32 = jnp.zeros((16,), jnp.float32)
    of32 = jnp.full((16,), 1.0, jnp.float32)

    @pl.loop(0, CHUNK)
    def _(i):
        for j in range(DH // 16):
            zero_v[i, pl.ds(16 * j, 16)] = zf32
        ones_v[i, :] = of32
        zdeg_v[i, :] = zf32

    node_base = sid * NODES_PER_TILE
    for k in range(NODES_PER_TILE // CHUNK):
        pltpu.sync_copy(zero_v, agg_sh.at[pl.ds(node_base + k * CHUNK, CHUNK)])
        pltpu.sync_copy(zdeg_v, deg_sh.at[pl.ds(node_base + k * CHUNK, CHUNK)])

    row_base = sid * ROWS_PER_TILE
    pltpu.sync_copy(src_hbm.at[pl.ds(row_base, ROWS_PER_TILE)], src_v)
    pltpu.sync_copy(dst_hbm.at[pl.ds(row_base, ROWS_PER_TILE)], dst_v)

    @pl.when(cid == 0)
    def _():
        _prime(x0_hbm, src_v, bufs, gsems)

    @pl.when(cid == 1)
    def _():
        _prime(x1_hbm, src_v, bufs, gsems)

    plsc.subcore_barrier()

    @pl.when(cid == 0)
    def _():
        _edge_loop(x0_hbm, src_v, dst_v, bufs, ones_v, agg_sh, deg_sh,
                   gsems, ssems, 0, DEG_SPLIT)

    @pl.when(cid == 1)
    def _():
        _edge_loop(x1_hbm, src_v, dst_v, bufs, ones_v, agg_sh, deg_sh,
                   gsems, ssems, DEG_SPLIT, ROWS_PER_TILE)

    plsc.subcore_barrier()

    wb_base = sid * WB_ROWS
    pltpu.sync_copy(agg_sh.at[pl.ds(wb_base, WB_ROWS)],
                    out_hbm.at[cid, pl.ds(wb_base, WB_ROWS)])
    pltpu.sync_copy(deg_sh.at[pl.ds(wb_base, WB_ROWS)],
                    deg_hbm.at[cid, pl.ds(wb_base, WB_ROWS)])

    @pl.when(sid == 0)
    def _():
        tail = N_NODES - NS * WB_ROWS
        pltpu.sync_copy(agg_sh.at[pl.ds(NS * WB_ROWS, tail)],
                        out_hbm.at[cid, pl.ds(NS * WB_ROWS, tail)])
        pltpu.sync_copy(deg_sh.at[pl.ds(NS * WB_ROWS, tail)],
                        deg_hbm.at[cid, pl.ds(NS * WB_ROWS, tail)])


_sc_agg = pl.kernel(
    _sc_agg_body,
    out_type=(jax.ShapeDtypeStruct((NC, N_NODES, DH), jnp.float32),
              jax.ShapeDtypeStruct((NC, N_NODES, DEG_W), jnp.float32)),
    mesh=plsc.VectorSubcoreMesh(core_axis_name="c", subcore_axis_name="s",
                                num_cores=NC, num_subcores=NS),
    scratch_types=(
        pltpu.VMEM((ROWS_PER_TILE, CHUNK), jnp.int32),
        pltpu.VMEM((ROWS_PER_TILE, CHUNK), jnp.int32),
        pltpu.VMEM((CHUNK, DH), jnp.float32),
        pltpu.VMEM((CHUNK, DH), jnp.float32),
        pltpu.VMEM((CHUNK, DH), jnp.float32),
        pltpu.VMEM((CHUNK, DH), jnp.float32),
        pltpu.VMEM((CHUNK, DH), jnp.float32),
        pltpu.VMEM((CHUNK, DEG_W), jnp.float32),
        pltpu.VMEM((CHUNK, DEG_W), jnp.float32),
        pltpu.VMEM_SHARED((N_NODES, DH), jnp.float32),
        pltpu.VMEM_SHARED((N_NODES, DEG_W), jnp.float32),
        pltpu.SemaphoreType.DMA,
        pltpu.SemaphoreType.DMA,
        pltpu.SemaphoreType.DMA,
        pltpu.SemaphoreType.DMA,
        pltpu.SemaphoreType.DMA,
        pltpu.SemaphoreType.DMA,
        pltpu.SemaphoreType.DMA,
        pltpu.SemaphoreType.DMA,
    ),
    compiler_params=pltpu.CompilerParams(use_tc_tiling_on_sc=False),
    name="sc_sage_agg",
)


def _tc_body(relu, p_ref, dp_ref, x_ref, wl_ref, b_ref, wr_ref, o_ref):
    agg = jnp.concatenate((p_ref[0], p_ref[1]), axis=1)
    deg = dp_ref[0, :, 0:1] + dp_ref[1, :, 0:1]
    a = agg / jnp.maximum(deg, 1.0)
    o = (jnp.dot(a, wl_ref[...], preferred_element_type=jnp.float32)
         + b_ref[...]
         + jnp.dot(x_ref[...], wr_ref[...], preferred_element_type=jnp.float32))
    o_ref[...] = jnp.maximum(o, 0.0) if relu else o


def _make_tc(relu):
    return pl.pallas_call(
        functools.partial(_tc_body, relu),
        out_shape=jax.ShapeDtypeStruct((N_NODES, D), jnp.float32),
    )


_tc1 = _make_tc(True)
_tc2 = _make_tc(False)


@jax.jit
def kernel(x, edge_index, W1l, b1, W1r, W2l, b2, W2r):
    src = edge_index[0].reshape(EROWS, CHUNK)
    dst = edge_index[1].reshape(EROWS, CHUNK)

    agg1, degp = _sc_agg(x[:, :DH], x[:, DH:], src, dst)
    h = _tc1(agg1, degp, x, W1l.T, b1.reshape(1, D), W1r.T)
    agg2, degp2 = _sc_agg(h[:, :DH], h[:, DH:], src, dst)
    out = _tc2(agg2, degp2, h, W2l.T, b2.reshape(1, D), W2r.T)
    return out


# speedup vs baseline: 12.1929x; 1.2010x over previous
import functools

import jax
import jax.numpy as jnp
from jax import lax
from jax.experimental import pallas as pl
from jax.experimental.pallas import tpu as pltpu
from jax.experimental.pallas import tpu_sc as plsc

N_NODES = 10000
N_EDGES = 320000
D = 128
DH = D // 2

NC = 2
NS = 16

CHUNK = 125
EROWS = N_EDGES // CHUNK
ROWS_PER_TILE = EROWS // NS
DEG_SPLIT = ROWS_PER_TILE // 2
NODES_PER_TILE = N_NODES // NS
WB_ROWS = NODES_PER_TILE // 8 * 8
DEG_W = 16


def _prime(nbuf, x_hbm, src_v, bufs, gsems):
    for b in range(nbuf):
        pltpu.async_copy(x_hbm.at[src_v.at[b]], bufs[b], gsems[b])


def _edge_loop(nbuf, x_hbm, src_v, dst_v, bufs, ones_v, agg_sh, deg_sh,
               gsems, ssems, deg_lo, deg_hi):
    @pl.loop(0, ROWS_PER_TILE, step=nbuf)
    def _(c):
        for b in range(nbuf):
            k = c + b
            pltpu.make_async_copy(x_hbm.at[src_v.at[k]], bufs[b],
                                  gsems[b]).wait()
            sdesc = pltpu.async_copy(bufs[b], agg_sh.at[dst_v.at[k]],
                                     ssems[b], add=True)

            if deg_sh is not None:
                @pl.when((k >= deg_lo) & (k < deg_hi))
                def _():
                    pltpu.sync_copy(ones_v, deg_sh.at[dst_v.at[k]], add=True)

            sdesc.wait()

            @pl.when(k + nbuf < ROWS_PER_TILE)
            def _():
                pltpu.async_copy(x_hbm.at[src_v.at[k + nbuf]], bufs[b],
                                 gsems[b])


def _sc_agg_body(with_deg, nbuf, *refs):
    if with_deg:
        (e_hbm, x0_hbm, x1_hbm, out_hbm, deg_hbm, src_v, dst_v,
         b0, b1, zero_v, ones_v, zdeg_v, agg_sh, deg_sh,
         g0, g1, s0, s1) = refs
        bufs, gsems, ssems = (b0, b1), (g0, g1), (s0, s1)
    else:
        (e_hbm, x0_hbm, x1_hbm, out_hbm, src_v, dst_v,
         b0, b1, b2, b3, zero_v, agg_sh,
         g0, g1, g2, g3, s0, s1, s2, s3) = refs
        bufs, gsems, ssems = (b0, b1, b2, b3), (g0, g1, g2, g3), (s0, s1, s2, s3)
        ones_v = zdeg_v = deg_sh = deg_hbm = None

    cid = lax.axis_index("c")
    sid = lax.axis_index("s")

    zf32 = jnp.zeros((16,), jnp.float32)
    of32 = jnp.full((16,), 1.0, jnp.float32)

    @pl.loop(0, CHUNK)
    def _(i):
        for j in range(DH // 16):
            zero_v[i, pl.ds(16 * j, 16)] = zf32
        if with_deg:
            ones_v[i, :] = of32
            zdeg_v[i, :] = zf32

    node_base = sid * NODES_PER_TILE
    for k in range(NODES_PER_TILE // CHUNK):
        pltpu.sync_copy(zero_v, agg_sh.at[pl.ds(node_base + k * CHUNK, CHUNK)])
        if with_deg:
            pltpu.sync_copy(zdeg_v, deg_sh.at[pl.ds(node_base + k * CHUNK, CHUNK)])

    row_base = sid * ROWS_PER_TILE
    pltpu.sync_copy(e_hbm.at[0, pl.ds(row_base, ROWS_PER_TILE)], src_v)
    pltpu.sync_copy(e_hbm.at[1, pl.ds(row_base, ROWS_PER_TILE)], dst_v)

    @pl.when(cid == 0)
    def _():
        _prime(nbuf, x0_hbm, src_v, bufs, gsems)

    @pl.when(cid == 1)
    def _():
        _prime(nbuf, x1_hbm, src_v, bufs, gsems)

    plsc.subcore_barrier()

    @pl.when(cid == 0)
    def _():
        _edge_loop(nbuf, x0_hbm, src_v, dst_v, bufs, ones_v, agg_sh, deg_sh,
                   gsems, ssems, 0, DEG_SPLIT)

    @pl.when(cid == 1)
    def _():
        _edge_loop(nbuf, x1_hbm, src_v, dst_v, bufs, ones_v, agg_sh, deg_sh,
                   gsems, ssems, DEG_SPLIT, ROWS_PER_TILE)

    plsc.subcore_barrier()

    wb_base = sid * WB_ROWS
    pltpu.sync_copy(agg_sh.at[pl.ds(wb_base, WB_ROWS)],
                    out_hbm.at[cid, pl.ds(wb_base, WB_ROWS)])
    if with_deg:
        pltpu.sync_copy(deg_sh.at[pl.ds(wb_base, WB_ROWS)],
                        deg_hbm.at[cid, pl.ds(wb_base, WB_ROWS)])

    @pl.when(sid == 0)
    def _():
        tail = N_NODES - NS * WB_ROWS
        pltpu.sync_copy(agg_sh.at[pl.ds(NS * WB_ROWS, tail)],
                        out_hbm.at[cid, pl.ds(NS * WB_ROWS, tail)])
        if with_deg:
            pltpu.sync_copy(deg_sh.at[pl.ds(NS * WB_ROWS, tail)],
                            deg_hbm.at[cid, pl.ds(NS * WB_ROWS, tail)])


def _make_sc(with_deg, nbuf):
    out_type = [jax.ShapeDtypeStruct((NC, N_NODES, DH), jnp.float32)]
    scratch = [
        pltpu.VMEM((ROWS_PER_TILE, CHUNK), jnp.int32),
        pltpu.VMEM((ROWS_PER_TILE, CHUNK), jnp.int32),
    ]
    scratch += [pltpu.VMEM((CHUNK, DH), jnp.float32)] * nbuf
    scratch.append(pltpu.VMEM((CHUNK, DH), jnp.float32))
    if with_deg:
        out_type.append(jax.ShapeDtypeStruct((NC, N_NODES, DEG_W), jnp.float32))
        scratch += [
            pltpu.VMEM((CHUNK, DEG_W), jnp.float32),
            pltpu.VMEM((CHUNK, DEG_W), jnp.float32),
        ]
    scratch.append(pltpu.VMEM_SHARED((N_NODES, DH), jnp.float32))
    if with_deg:
        scratch.append(pltpu.VMEM_SHARED((N_NODES, DEG_W), jnp.float32))
    scratch += [pltpu.SemaphoreType.DMA] * (2 * nbuf)

    return pl.kernel(
        functools.partial(_sc_agg_body, with_deg, nbuf),
        out_type=tuple(out_type) if with_deg else out_type[0],
        mesh=plsc.VectorSubcoreMesh(core_axis_name="c", subcore_axis_name="s",
                                    num_cores=NC, num_subcores=NS),
        scratch_types=tuple(scratch),
        compiler_params=pltpu.CompilerParams(use_tc_tiling_on_sc=False),
        name="sc_sage_agg_deg" if with_deg else "sc_sage_agg",
    )


_sc_agg_deg = _make_sc(True, 2)
_sc_agg2 = _make_sc(False, 4)


def _tc1_body(p_ref, dp_ref, x_ref, wl_ref, b_ref, wr_ref, h0_ref, h1_ref):
    agg = jnp.concatenate((p_ref[0], p_ref[1]), axis=1)
    deg = dp_ref[0, :, 0:1] + dp_ref[1, :, 0:1]
    a = agg / jnp.maximum(deg, 1.0)
    h = (jnp.dot(a, wl_ref[...], preferred_element_type=jnp.float32)
         + b_ref[...]
         + jnp.dot(x_ref[...], wr_ref[...], preferred_element_type=jnp.float32))
    h = jnp.maximum(h, 0.0)
    h0_ref[...] = h[:, :DH]
    h1_ref[...] = h[:, DH:]


def _tc2_body(p_ref, dp_ref, h0_ref, h1_ref, wl_ref, b_ref, wr_ref, o_ref):
    agg = jnp.concatenate((p_ref[0], p_ref[1]), axis=1)
    deg = dp_ref[0, :, 0:1] + dp_ref[1, :, 0:1]
    a = agg / jnp.maximum(deg, 1.0)
    xr = (jnp.dot(h0_ref[...], wr_ref[:DH, :], preferred_element_type=jnp.float32)
          + jnp.dot(h1_ref[...], wr_ref[DH:, :], preferred_element_type=jnp.float32))
    o_ref[...] = (jnp.dot(a, wl_ref[...], preferred_element_type=jnp.float32)
                  + b_ref[...] + xr)


_tc1 = pl.pallas_call(
    _tc1_body,
    out_shape=(jax.ShapeDtypeStruct((N_NODES, DH), jnp.float32),
               jax.ShapeDtypeStruct((N_NODES, DH), jnp.float32)),
)

_tc2 = pl.pallas_call(
    _tc2_body,
    out_shape=jax.ShapeDtypeStruct((N_NODES, D), jnp.float32),
)


@jax.jit
def kernel(x, edge_index, W1l, b1, W1r, W2l, b2, W2r):
    e3 = edge_index.reshape(2, EROWS, CHUNK)

    agg1, degp = _sc_agg_deg(e3, x[:, :DH], x[:, DH:])
    h0, h1 = _tc1(agg1, degp, x, W1l.T, b1.reshape(1, D), W1r.T)
    agg2 = _sc_agg2(e3, h0, h1)
    out = _tc2(agg2, degp, h0, h1, W2l.T, b2.reshape(1, D), W2r.T)
    return out
